# baseline (speedup 1.0000x reference)
import functools

import jax
import jax.numpy as jnp
from jax.experimental import pallas as pl
from jax.experimental.pallas import tpu as pltpu

WS = 8
TOK = WS * WS


def _attn_body(nwin, x_ref, wq_ref, bq_ref, wk_ref, bk_ref, wv_ref, bv_ref,
               o_ref, q_ref, k_ref, v_ref):
    xb = x_ref[...]
    q_ref[...] = jnp.dot(xb, wq_ref[...],
                         preferred_element_type=jnp.float32) + bq_ref[...]
    k_ref[...] = jnp.dot(xb, wk_ref[...],
                         preferred_element_type=jnp.float32) + bk_ref[...]
    v_ref[...] = jnp.dot(xb, wv_ref[...],
                         preferred_element_type=jnp.float32) + bv_ref[...]

    def win(i, _):
        qw = q_ref[pl.ds(i * TOK, TOK), :]
        kw = k_ref[pl.ds(i * TOK, TOK), :]
        vw = v_ref[pl.ds(i * TOK, TOK), :]
        a = jax.lax.dot_general(qw, kw, (((1,), (1,)), ((), ())),
                                preferred_element_type=jnp.float32)
        a = a - jnp.max(a, axis=-1, keepdims=True)
        e = jnp.exp(a)
        p = e / jnp.sum(e, axis=-1, keepdims=True)
        ow = jnp.dot(p, vw, preferred_element_type=jnp.float32)
        o_ref[pl.ds(i * TOK, TOK), :] = ow
        return 0

    jax.lax.fori_loop(0, nwin, win, 0)


def kernel(x, Wv, bv, Wq, bq, Wk, bk):
    b, c, h, w = x.shape
    hn, wn = h // WS, w // WS
    nwin_total = b * hn * wn

    xw = x.reshape(b, c, hn, WS, wn, WS)
    xw = jnp.transpose(xw, (0, 2, 4, 3, 5, 1))
    xw = xw.reshape(nwin_total * TOK, c)

    nwin_blk = min(64, nwin_total)
    rows = nwin_blk * TOK
    grid = (nwin_total // nwin_blk,)

    wspec = pl.BlockSpec((c, c), lambda i: (0, 0))
    bspec = pl.BlockSpec((1, c), lambda i: (0, 0))
    out = pl.pallas_call(
        functools.partial(_attn_body, nwin_blk),
        grid=grid,
        in_specs=[
            pl.BlockSpec((rows, c), lambda i: (i, 0)),
            wspec, bspec, wspec, bspec, wspec, bspec,
        ],
        out_specs=pl.BlockSpec((rows, c), lambda i: (i, 0)),
        out_shape=jax.ShapeDtypeStruct((nwin_total * TOK, c), jnp.float32),
        scratch_shapes=[pltpu.VMEM((rows, c), jnp.float32)] * 3,
    )(xw, Wq.T, bq[None, :], Wk.T, bk[None, :], Wv.T, bv[None, :])

    out = out.reshape(b, hn, wn, WS, WS, c)
    out = jnp.transpose(out, (0, 5, 1, 3, 2, 4)).reshape(b, c, h, w)
    return out


# speedup vs baseline: 2.0641x; 2.0641x over previous
import functools

import jax
import jax.numpy as jnp
from jax.experimental import pallas as pl
from jax.experimental.pallas import tpu as pltpu

WS = 8
PACK = 2


def _body(wn, x_ref, wq_ref, bq_ref, wk_ref, bk_ref, wv_ref, bv_ref,
          o_ref, q_ref, k_ref, v_ref, t_ref):
    w = wn * WS
    xb = x_ref[0, :, 0]
    for dh in range(WS):
        xs = xb[:, dh, :]
        cd = (((0,), (0,)), ((), ()))
        q_ref[dh] = jax.lax.dot_general(
            xs, wq_ref[...], cd, preferred_element_type=jnp.float32) + bq_ref[...]
        k_ref[dh] = jax.lax.dot_general(
            xs, wk_ref[...], cd, preferred_element_type=jnp.float32) + bk_ref[...]
        v_ref[dh] = jax.lax.dot_general(
            xs, wv_ref[...], cd, preferred_element_type=jnp.float32) + bv_ref[...]

    rows = WS * WS * PACK
    c = wq_ref.shape[1]
    r = jax.lax.broadcasted_iota(jnp.int32, (rows, rows), 0)
    s = jax.lax.broadcasted_iota(jnp.int32, (rows, rows), 1)
    mask = ((r // WS) % PACK) == ((s // WS) % PACK)

    def pack_step(i, _):
        sl = (slice(None), pl.ds(i * WS * PACK, WS * PACK), slice(None))
        qp = q_ref[sl].reshape(rows, c)
        kp = k_ref[sl].reshape(rows, c)
        vp = v_ref[sl].reshape(rows, c)
        a = jax.lax.dot_general(qp, kp, (((1,), (1,)), ((), ())),
                                preferred_element_type=jnp.float32)
        a = jnp.where(mask, a, -1e30)
        a = a - jnp.max(a, axis=-1, keepdims=True)
        e = jnp.exp(a)
        p = e / jnp.sum(e, axis=-1, keepdims=True)
        ow = jnp.dot(p, vp, preferred_element_type=jnp.float32)
        t_ref[sl] = ow.reshape(WS, WS * PACK, c)
        return 0

    jax.lax.fori_loop(0, wn // PACK, pack_step, 0)

    for dh in range(WS):
        o_ref[0, :, 0, dh, :] = jnp.swapaxes(t_ref[dh], 0, 1)


def kernel(x, Wv, bv, Wq, bq, Wk, bk):
    b, c, h, w = x.shape
    hn, wn = h // WS, w // WS

    wspec = pl.BlockSpec((c, c), lambda i, j: (0, 0))
    bspec = pl.BlockSpec((1, c), lambda i, j: (0, 0))
    xspec = pl.BlockSpec((1, c, 1, WS, w), lambda i, j: (i, 0, j, 0, 0))

    out = pl.pallas_call(
        functools.partial(_body, wn),
        grid=(b, hn),
        in_specs=[xspec, wspec, bspec, wspec, bspec, wspec, bspec],
        out_specs=xspec,
        out_shape=jax.ShapeDtypeStruct((b, c, hn, WS, w), jnp.float32),
        scratch_shapes=[pltpu.VMEM((WS, w, c), jnp.float32)] * 4,
    )(x.reshape(b, c, hn, WS, w), Wq.T, bq[None, :], Wk.T, bk[None, :],
      Wv.T, bv[None, :])
    return out.reshape(b, c, h, w)


# speedup vs baseline: 2.0678x; 1.0018x over previous
import functools

import jax
import jax.numpy as jnp
from jax.experimental import pallas as pl
from jax.experimental.pallas import tpu as pltpu

WS = 8
PACK = 2


def _body(wn, x_ref, wq_ref, bq_ref, wk_ref, bk_ref, wv_ref, bv_ref,
          o_ref, q_ref, k_ref, v_ref, t_ref):
    w = wn * WS
    xb = x_ref[0, :, 0]
    for dh in range(WS):
        xs = xb[:, dh, :]
        cd = (((0,), (0,)), ((), ()))
        q_ref[dh] = jax.lax.dot_general(
            xs, wq_ref[...], cd, preferred_element_type=jnp.float32) + bq_ref[...]
        k_ref[dh] = jax.lax.dot_general(
            xs, wk_ref[...], cd, preferred_element_type=jnp.float32) + bk_ref[...]
        v_ref[dh] = jax.lax.dot_general(
            xs, wv_ref[...], cd, preferred_element_type=jnp.float32) + bv_ref[...]

    rows = WS * WS * PACK
    c = wq_ref.shape[1]
    r = jax.lax.broadcasted_iota(jnp.int32, (rows, rows), 0)
    s = jax.lax.broadcasted_iota(jnp.int32, (rows, rows), 1)
    mask = ((r // WS) % PACK) == ((s // WS) % PACK)

    def pack_step(i, _):
        sl = (slice(None), pl.ds(i * WS * PACK, WS * PACK), slice(None))
        qp = q_ref[sl].reshape(rows, c)
        kp = k_ref[sl].reshape(rows, c)
        vp = v_ref[sl].reshape(rows, c)
        a = jax.lax.dot_general(qp, kp, (((1,), (1,)), ((), ())),
                                preferred_element_type=jnp.float32)
        a = jnp.where(mask, a, -1e30)
        a = a - jnp.max(a, axis=-1, keepdims=True)
        e = jnp.exp(a)
        p = e / jnp.sum(e, axis=-1, keepdims=True)
        ow = jnp.dot(p, vp, preferred_element_type=jnp.float32)
        t_ref[sl] = ow.reshape(WS, WS * PACK, c)
        return 0

    jax.lax.fori_loop(0, wn // PACK, pack_step, 0)

    for dh in range(WS):
        o_ref[0, :, 0, dh, :] = jnp.swapaxes(t_ref[dh], 0, 1)


def kernel(x, Wv, bv, Wq, bq, Wk, bk):
    b, c, h, w = x.shape
    hn, wn = h // WS, w // WS

    wspec = pl.BlockSpec((c, c), lambda i, j: (0, 0))
    bspec = pl.BlockSpec((1, c), lambda i, j: (0, 0))
    xspec = pl.BlockSpec((1, c, 1, WS, w), lambda i, j: (i, 0, j, 0, 0))

    out = pl.pallas_call(
        functools.partial(_body, wn),
        grid=(b, hn),
        in_specs=[xspec, wspec, bspec, wspec, bspec, wspec, bspec],
        out_specs=xspec,
        out_shape=jax.ShapeDtypeStruct((b, c, hn, WS, w), jnp.float32),
        scratch_shapes=[pltpu.VMEM((WS, w, c), jnp.float32)] * 4,
        compiler_params=pltpu.CompilerParams(
            dimension_semantics=("parallel", "parallel")),
    )(x.reshape(b, c, hn, WS, w), Wq.T, bq[None, :], Wk.T, bk[None, :],
      Wv.T, bv[None, :])
    return out.reshape(b, c, h, w)


# speedup vs baseline: 5.4032x; 2.6130x over previous
import functools

import jax
import jax.numpy as jnp
from jax.experimental import pallas as pl
from jax.experimental.pallas import tpu as pltpu

WS = 8
PACK = 2


def _body(wn, x_ref, wq_ref, bq_ref, wk_ref, bk_ref, wv_ref, bv_ref,
          o_ref, q_ref, k_ref, v_ref, t_ref):
    w = wn * WS
    xb = x_ref[0, :, 0]
    for dh in range(WS):
        xsT = jnp.swapaxes(xb[:, dh, :], 0, 1)
        q_ref[dh] = jnp.dot(xsT, wq_ref[...],
                            preferred_element_type=jnp.float32) + bq_ref[...]
        k_ref[dh] = jnp.dot(xsT, wk_ref[...],
                            preferred_element_type=jnp.float32) + bk_ref[...]
        v_ref[dh] = jnp.dot(xsT, wv_ref[...],
                            preferred_element_type=jnp.float32) + bv_ref[...]

    rows = WS * WS * PACK
    c = wq_ref.shape[1]
    r = jax.lax.broadcasted_iota(jnp.int32, (rows, rows), 0)
    s = jax.lax.broadcasted_iota(jnp.int32, (rows, rows), 1)
    bias = jnp.where(((r // WS) % PACK) == ((s // WS) % PACK), 0.0, -1e30)

    for i in range(wn // PACK):
        sl = (slice(None), pl.ds(i * WS * PACK, WS * PACK), slice(None))
        qp = q_ref[sl].reshape(rows, c)
        kp = k_ref[sl].reshape(rows, c)
        vp = v_ref[sl].reshape(rows, c)
        a = jax.lax.dot_general(qp, kp, (((1,), (1,)), ((), ())),
                                preferred_element_type=jnp.float32) + bias
        a = a - jnp.max(a, axis=-1, keepdims=True)
        e = jnp.exp(a)
        p = e / jnp.sum(e, axis=-1, keepdims=True)
        ow = jnp.dot(p, vp, preferred_element_type=jnp.float32)
        t_ref[sl] = ow.reshape(WS, WS * PACK, c)

    for dh in range(WS):
        o_ref[0, :, 0, dh, :] = jnp.swapaxes(t_ref[dh], 0, 1)


def kernel(x, Wv, bv, Wq, bq, Wk, bk):
    b, c, h, w = x.shape
    hn, wn = h // WS, w // WS

    wspec = pl.BlockSpec((c, c), lambda i, j: (0, 0))
    bspec = pl.BlockSpec((1, c), lambda i, j: (0, 0))
    xspec = pl.BlockSpec((1, c, 1, WS, w), lambda i, j: (i, 0, j, 0, 0))

    out = pl.pallas_call(
        functools.partial(_body, wn),
        grid=(b, hn),
        in_specs=[xspec, wspec, bspec, wspec, bspec, wspec, bspec],
        out_specs=xspec,
        out_shape=jax.ShapeDtypeStruct((b, c, hn, WS, w), jnp.float32),
        scratch_shapes=[pltpu.VMEM((WS, w, c), jnp.float32)] * 4,
        compiler_params=pltpu.CompilerParams(
            dimension_semantics=("parallel", "parallel")),
    )(x.reshape(b, c, hn, WS, w), Wq.T, bq[None, :], Wk.T, bk[None, :],
      Wv.T, bv[None, :])
    return out.reshape(b, c, h, w)


# speedup vs baseline: 5.8667x; 1.0858x over previous
import functools

import jax
import jax.numpy as jnp
from jax.experimental import pallas as pl
from jax.experimental.pallas import tpu as pltpu

WS = 8
PACK = 2
PAD = 256


def _body(wn, x_ref, wqkv_ref, bqkv_ref, o_ref, qkv_ref, t_ref):
    w = wn * WS
    c = t_ref.shape[2]
    for dh in range(WS):
        xsT = jnp.swapaxes(x_ref[0, :, 0, dh, :], 0, 1)
        qkv_ref[dh] = jnp.dot(xsT, wqkv_ref[...],
                              preferred_element_type=jnp.float32) + bqkv_ref[...]

    rows = WS * WS * PACK
    r = jax.lax.broadcasted_iota(jnp.int32, (rows, rows), 0)
    s = jax.lax.broadcasted_iota(jnp.int32, (rows, rows), 1)
    bias = jnp.where(((r // WS) % PACK) == ((s // WS) % PACK), 0.0, -1e30)

    for i in range(wn // PACK):
        ds = pl.ds(i * WS * PACK, WS * PACK)
        qp = qkv_ref[:, ds, 0:PAD].reshape(rows, PAD)
        kp = qkv_ref[:, ds, PAD:2 * PAD].reshape(rows, PAD)
        vp = qkv_ref[:, ds, 2 * PAD:2 * PAD + c].reshape(rows, c)
        a = jax.lax.dot_general(qp, kp, (((1,), (1,)), ((), ())),
                                preferred_element_type=jnp.float32) + bias
        a = a - jnp.max(a, axis=-1, keepdims=True)
        e = jnp.exp(a)
        p = e / jnp.sum(e, axis=-1, keepdims=True)
        ow = jnp.dot(p, vp, preferred_element_type=jnp.float32)
        t_ref[:, ds, :] = ow.reshape(WS, WS * PACK, c)

    for dh in range(WS):
        o_ref[0, :, 0, dh, :] = jnp.swapaxes(t_ref[dh], 0, 1)


def kernel(x, Wv, bv, Wq, bq, Wk, bk):
    b, c, h, w = x.shape
    hn, wn = h // WS, w // WS

    wqkv = jnp.zeros((c, 2 * PAD + c), jnp.float32)
    wqkv = wqkv.at[:, 0:c].set(Wq.T)
    wqkv = wqkv.at[:, PAD:PAD + c].set(Wk.T)
    wqkv = wqkv.at[:, 2 * PAD:2 * PAD + c].set(Wv.T)
    bqkv = jnp.zeros((1, 2 * PAD + c), jnp.float32)
    bqkv = bqkv.at[:, 0:c].set(bq[None, :])
    bqkv = bqkv.at[:, PAD:PAD + c].set(bk[None, :])
    bqkv = bqkv.at[:, 2 * PAD:2 * PAD + c].set(bv[None, :])

    wspec = pl.BlockSpec((c, 2 * PAD + c), lambda i, j: (0, 0))
    bspec = pl.BlockSpec((1, 2 * PAD + c), lambda i, j: (0, 0))
    xspec = pl.BlockSpec((1, c, 1, WS, w), lambda i, j: (i, 0, j, 0, 0))

    out = pl.pallas_call(
        functools.partial(_body, wn),
        grid=(b, hn),
        in_specs=[xspec, wspec, bspec],
        out_specs=xspec,
        out_shape=jax.ShapeDtypeStruct((b, c, hn, WS, w), jnp.float32),
        scratch_shapes=[pltpu.VMEM((WS, wn * WS, 2 * PAD + c), jnp.float32),
                        pltpu.VMEM((WS, wn * WS, c), jnp.float32)],
        compiler_params=pltpu.CompilerParams(
            dimension_semantics=("parallel", "parallel")),
    )(x.reshape(b, c, hn, WS, w), wqkv, bqkv)
    return out.reshape(b, c, h, w)
